# Initial kernel scaffold; baseline (speedup 1.0000x reference)
import jax
import jax.numpy as jnp
from jax.experimental import pallas as pl


def kernel(V_Adap):
    raise NotImplementedError("write your pallas kernel here")


# speedup vs baseline: 1.1147x; 1.1147x over previous
import jax
import jax.numpy as jnp
from jax.experimental import pallas as pl

B, N = 8, 1024
ROWS = B * N
BLK_R = 512


def _body(v_ref, idx_ref, attr_ref):
    i = pl.program_id(0)
    attr_ref[...] = jax.nn.sigmoid(v_ref[...])
    base = (i * BLK_R) % N
    r = jax.lax.broadcasted_iota(jnp.int32, (BLK_R, N), 0) + base
    c = jax.lax.broadcasted_iota(jnp.int32, (BLK_R, N), 1)
    idx_ref[0] = r
    idx_ref[1] = c


def kernel(V_Adap):
    v2d = V_Adap.reshape(ROWS, N)
    grid = (ROWS // BLK_R,)
    edge_index3, attr2 = pl.pallas_call(
        _body,
        grid=grid,
        in_specs=[pl.BlockSpec((BLK_R, N), lambda i: (i, 0))],
        out_specs=[
            pl.BlockSpec((2, BLK_R, N), lambda i: (0, i, 0)),
            pl.BlockSpec((BLK_R, N), lambda i: (i, 0)),
        ],
        out_shape=[
            jax.ShapeDtypeStruct((2, ROWS, N), jnp.int32),
            jax.ShapeDtypeStruct((ROWS, N), jnp.float32),
        ],
    )(v2d)
    return edge_index3.reshape(2, ROWS * N), attr2.reshape(ROWS * N)


# speedup vs baseline: 1.1789x; 1.0576x over previous
import jax
import jax.numpy as jnp
from jax.experimental import pallas as pl

B, N = 8, 1024
E = B * N * N
NBLK = 16
C = E // NBLK


def _body(v_ref, idx_ref, attr_ref):
    i = pl.program_id(0)
    attr_ref[...] = jax.nn.sigmoid(v_ref[...])
    j = jax.lax.broadcasted_iota(jnp.int32, (C,), 0)
    col = j & (N - 1)
    row = ((i * C + j) >> 10) & (N - 1)
    idx_ref[0, :] = row
    idx_ref[1, :] = col


def kernel(V_Adap):
    v_flat = V_Adap.reshape(E)
    edge_index, edge_attr = pl.pallas_call(
        _body,
        grid=(NBLK,),
        in_specs=[pl.BlockSpec((C,), lambda i: (i,))],
        out_specs=[
            pl.BlockSpec((2, C), lambda i: (0, i)),
            pl.BlockSpec((C,), lambda i: (i,)),
        ],
        out_shape=[
            jax.ShapeDtypeStruct((2, E), jnp.int32),
            jax.ShapeDtypeStruct((E,), jnp.float32),
        ],
    )(v_flat)
    return edge_index, edge_attr


# speedup vs baseline: 1.7184x; 1.4576x over previous
import functools

import jax
import jax.numpy as jnp
from jax import lax
from jax.experimental import pallas as pl
from jax.experimental.pallas import tpu as pltpu
from jax.experimental.pallas import tpu_sc as plsc

B, N = 8, 1024
P = N * N
E = B * P

NC, NS = 2, 16
NW = NC * NS
S = P // NW
VPB = N // 16


def _sc_body(out_hbm, row_v, col_v, sem):
    c = lax.axis_index("c")
    s = lax.axis_index("s")
    wid = s * NC + c
    lane = lax.iota(jnp.int32, 16)

    base = wid * (S // N)

    def fill(t, _):
        rval = jnp.broadcast_to(base + (t >> 3), (16,)).astype(jnp.int32)
        for u in range(8):
            m = t * 8 + u
            row_v[pl.ds(m * 16, 16)] = rval
            col_v[pl.ds(m * 16, 16)] = lane + ((m * 16) & (N - 1))
        return 0

    lax.fori_loop(0, S // (16 * 8), fill, 0)

    copies = []
    for b in range(B):
        off = b * P + wid * S
        copies.append(pltpu.make_async_copy(row_v, out_hbm.at[0, pl.ds(off, S)], sem))
        copies.append(pltpu.make_async_copy(col_v, out_hbm.at[1, pl.ds(off, S)], sem))
    for cp in copies:
        cp.start()
    for cp in copies:
        cp.wait()


_sc_edge_index = functools.partial(
    pl.kernel,
    out_type=jax.ShapeDtypeStruct((2, E), jnp.int32),
    mesh=plsc.VectorSubcoreMesh(core_axis_name="c", subcore_axis_name="s"),
    scratch_types=[
        pltpu.VMEM((S,), jnp.int32),
        pltpu.VMEM((S,), jnp.int32),
        pltpu.SemaphoreType.DMA,
    ],
)(_sc_body)


NBLK = 16
C = E // NBLK


def _tc_body(v_ref, attr_ref):
    attr_ref[...] = 0.5 * jnp.tanh(0.5 * v_ref[...]) + 0.5


def kernel(V_Adap):
    edge_index = _sc_edge_index()
    v_flat = V_Adap.reshape(E)
    edge_attr = pl.pallas_call(
        _tc_body,
        grid=(NBLK,),
        in_specs=[pl.BlockSpec((C,), lambda i: (i,))],
        out_specs=pl.BlockSpec((C,), lambda i: (i,)),
        out_shape=jax.ShapeDtypeStruct((E,), jnp.float32),
    )(v_flat)
    return edge_index, edge_attr


# speedup vs baseline: 2.4666x; 1.4354x over previous
import functools

import jax
import jax.numpy as jnp
from jax import lax
from jax.experimental import pallas as pl
from jax.experimental.pallas import tpu as pltpu
from jax.experimental.pallas import tpu_sc as plsc

B, N = 8, 1024
P = N * N
E = B * P

NC, NS = 2, 16
NW = NC * NS
S = P // NW
VPB = N // 16


def _sc_body(out_hbm, row_v, col_v, sem):
    c = lax.axis_index("c")
    s = lax.axis_index("s")
    wid = s * NC + c
    lane = lax.iota(jnp.int32, 16)

    base = wid * (S // N)

    def fill(t, _):
        rval = jnp.broadcast_to(base + (t >> 3), (16,)).astype(jnp.int32)
        for u in range(8):
            m = t * 8 + u
            row_v[pl.ds(m * 16, 16)] = rval
            col_v[pl.ds(m * 16, 16)] = lane + ((m * 16) & (N - 1))
        return 0

    lax.fori_loop(0, S // (16 * 8), fill, 0)

    copies = []
    for b in range(B):
        off = b * P + wid * S
        copies.append(pltpu.make_async_copy(row_v, out_hbm.at[0, pl.ds(off, S)], sem))
        copies.append(pltpu.make_async_copy(col_v, out_hbm.at[1, pl.ds(off, S)], sem))
    for cp in copies:
        cp.start()
    for cp in copies:
        cp.wait()


_sc_edge_index = functools.partial(
    pl.kernel,
    out_type=jax.ShapeDtypeStruct((2, E), jnp.int32),
    mesh=plsc.VectorSubcoreMesh(core_axis_name="c", subcore_axis_name="s"),
    scratch_types=[
        pltpu.VMEM((S,), jnp.int32),
        pltpu.VMEM((S,), jnp.int32),
        pltpu.SemaphoreType.DMA,
    ],
)(_sc_body)


NBLK = 16
C = E // NBLK
BLK_R = C // N


def _tc_body(v_ref, attr_ref):
    attr_ref[...] = (0.5 * jnp.tanh(0.5 * v_ref[...]) + 0.5).reshape(C)


def kernel(V_Adap):
    edge_index = _sc_edge_index()
    v2d = V_Adap.reshape(B * N, N)
    edge_attr = pl.pallas_call(
        _tc_body,
        grid=(NBLK,),
        in_specs=[pl.BlockSpec((BLK_R, N), lambda i: (i, 0))],
        out_specs=pl.BlockSpec((C,), lambda i: (i,)),
        out_shape=jax.ShapeDtypeStruct((E,), jnp.float32),
    )(v2d)
    return edge_index, edge_attr
